# baseline (speedup 1.0000x reference)
import functools

import jax
import jax.numpy as jnp
from jax import lax
from jax.experimental import pallas as pl
from jax.experimental.pallas import tpu as pltpu
from jax.experimental.pallas import tpu_sc as plsc

EMB_N = 1000000
EMB_D = 32
B = 16384

_NC = 2
_NS = 16
_NW = _NC * _NS
_B_PER_W = B // _NW

_TILE_COLS = (EMB_N + 127) // 128
_R = EMB_D // 8

_W_PER_R = _NW // _R
_COLS_PER_W = -(-_TILE_COLS // _W_PER_R)
_CHUNK = 16


_PADN = _TILE_COLS * 128
_SLAB = 977


@functools.partial(
    pl.kernel,
    out_type=jax.ShapeDtypeStruct((_R, 8, _PADN), jnp.float32),
    mesh=plsc.VectorSubcoreMesh(core_axis_name="c", subcore_axis_name="s"),
    scratch_types=[],
    compiler_params=pltpu.CompilerParams(use_tc_tiling_on_sc=True),
)
def _sc_relabel(tableT_hbm, raw_hbm, *, _n=_SLAB * 128):
    wid = lax.axis_index("s") * _NC + lax.axis_index("c")
    r = wid // _W_PER_R
    slot = wid % _W_PER_R
    lo = jnp.minimum(slot * _SLAB, _TILE_COLS - _SLAB) * 128
    lo = pl.multiple_of(lo, 128)
    pltpu.sync_copy(
        tableT_hbm.at[pl.ds(r * 8, 8), pl.ds(lo, _n)],
        raw_hbm.at[r, :, pl.ds(lo, _n)],
    )


_L = 16


@functools.partial(
    pl.kernel,
    out_type=jax.ShapeDtypeStruct((_R, 8, B), jnp.float32),
    mesh=plsc.VectorSubcoreMesh(core_axis_name="c", subcore_axis_name="s"),
    scratch_types=[
        pltpu.VMEM((_B_PER_W,), jnp.int32),
        pltpu.VMEM((_R, 8, _B_PER_W), jnp.int32),
        pltpu.VMEM((_R, 8, _B_PER_W), jnp.float32),
        pltpu.SemaphoreType.DMA,
    ],
    compiler_params=pltpu.CompilerParams(use_tc_tiling_on_sc=False),
)
def _sc_gather(idx_hbm, raw_hbm, out_hbm, idx_v, addr_v, buf_v, sem):
    wid = lax.axis_index("s") * _NC + lax.axis_index("c")
    base = wid * _B_PER_W
    pltpu.sync_copy(idx_hbm.at[pl.ds(base, _B_PER_W)], idx_v)

    def addr_body(q, _):
        iv = idx_v[pl.ds(q * _L, _L)]
        s = (iv >> 7) * 1024 + (iv & 127)
        for rr in range(_R * 8):
            r_, rr_ = rr // 8, rr % 8
            addr_v[r_, rr_, pl.ds(q * _L, _L)] = s + (
                (r_ * _TILE_COLS * 8 + rr_) * 128
            )
        return ()

    lax.fori_loop(0, _B_PER_W // _L, addr_body, ())

    copies = []
    for rr in range(_R * 8):
        r_, rr_ = rr // 8, rr % 8
        for k in range(_B_PER_W // 128):
            cp = pltpu.make_async_copy(
                raw_hbm.at[addr_v.at[r_, rr_, pl.ds(k * 128, 128)]],
                buf_v.at[r_, rr_, pl.ds(k * 128, 128)],
                sem,
            )
            cp.start()
            copies.append(cp)
    for cp in copies:
        cp.wait()
    pltpu.sync_copy(buf_v, out_hbm.at[:, :, pl.ds(base, _B_PER_W)])


def kernel(idx, table):
    raw = _sc_relabel(table.T)
    flat = (
        raw.reshape(_R, 8, _TILE_COLS, 128)
        .transpose(0, 2, 1, 3)
        .reshape(-1)
    )
    out3 = _sc_gather(idx.astype(jnp.int32), flat)
    return out3.reshape(EMB_D, B).T


# speedup vs baseline: 23.4618x; 23.4618x over previous
import functools

import jax
import jax.numpy as jnp
from jax import lax
from jax.experimental import pallas as pl
from jax.experimental.pallas import tpu as pltpu
from jax.experimental.pallas import tpu_sc as plsc

EMB_N = 1000000
EMB_D = 32
B = 16384

_NC = 2
_NS = 16
_NW = _NC * _NS
_B_PER_W = B // _NW

_TILE_COLS = (EMB_N + 127) // 128
_R = EMB_D // 8

_W_PER_R = _NW // _R
_COLS_PER_W = -(-_TILE_COLS // _W_PER_R)
_CHUNK = 16


@functools.partial(
    pl.kernel,
    out_type=jax.ShapeDtypeStruct((_R, _TILE_COLS, 8, 128), jnp.float32),
    mesh=plsc.VectorSubcoreMesh(core_axis_name="c", subcore_axis_name="s"),
    scratch_types=[
        pltpu.VMEM((2, 8, _CHUNK * 128), jnp.float32),
        pltpu.SemaphoreType.DMA,
        pltpu.SemaphoreType.DMA,
    ],
    compiler_params=pltpu.CompilerParams(use_tc_tiling_on_sc=True),
)
def _sc_relabel(tableT_hbm, raw_hbm, buf_v, sem_r, sem_w):
    wid = lax.axis_index("s") * _NC + lax.axis_index("c")
    r = wid // _W_PER_R
    c_lo = (wid % _W_PER_R) * _COLS_PER_W
    c_hi = jnp.minimum(c_lo + _COLS_PER_W, _TILE_COLS)
    n_full = (c_hi - c_lo) // _CHUNK

    def read_chunk(s, parity):
        c0 = c_lo + s * _CHUNK
        return pltpu.make_async_copy(
            tableT_hbm.at[
                pl.ds(r * 8, 8),
                pl.ds(pl.multiple_of(c0 * 128, 128), _CHUNK * 128),
            ],
            buf_v.at[parity],
            sem_r,
        )

    def tile_write(s, parity, j):
        c0 = c_lo + s * _CHUNK
        return pltpu.make_async_copy(
            buf_v.at[parity, :, pl.ds(j * 128, 128)],
            raw_hbm.at[r, c0 + j],
            sem_w,
        )

    @pl.when(n_full > 0)
    def _prime():
        read_chunk(0, 0).start()

    def step(s, _):
        parity = lax.rem(s, 2)

        @pl.when(s >= 2)
        def _drain():
            for j in range(_CHUNK):
                tile_write(s - 2, parity, j).wait()

        read_chunk(s, parity).wait()

        @pl.when(s + 1 < n_full)
        def _next():
            read_chunk(s + 1, 1 - parity).start()

        for j in range(_CHUNK):
            tile_write(s, parity, j).start()
        return ()

    lax.fori_loop(0, n_full, step, ())

    @pl.when(n_full >= 1)
    def _drain_last():
        for j in range(_CHUNK):
            tile_write(n_full - 1, lax.rem(n_full - 1, 2), j).wait()

    @pl.when(n_full >= 2)
    def _drain_prev():
        for j in range(_CHUNK):
            tile_write(n_full - 2, lax.rem(n_full - 2, 2), j).wait()

    def col_body(c, _):
        pltpu.sync_copy(
            tableT_hbm.at[
                pl.ds(r * 8, 8),
                pl.ds(pl.multiple_of(c * 128, 128), 128),
            ],
            buf_v.at[0, :, pl.ds(0, 128)],
        )
        pltpu.sync_copy(
            buf_v.at[0, :, pl.ds(0, 128)],
            raw_hbm.at[r, c],
        )
        return ()

    lax.fori_loop(c_lo + n_full * _CHUNK, c_hi, col_body, ())


_L = 16


@functools.partial(
    pl.kernel,
    out_type=jax.ShapeDtypeStruct((_R, 8, B), jnp.float32),
    mesh=plsc.VectorSubcoreMesh(core_axis_name="c", subcore_axis_name="s"),
    scratch_types=[
        pltpu.VMEM((_B_PER_W,), jnp.int32),
        pltpu.VMEM((_R, 8, _B_PER_W), jnp.int32),
        pltpu.VMEM((_R, 8, _B_PER_W), jnp.float32),
        pltpu.SemaphoreType.DMA,
    ],
    compiler_params=pltpu.CompilerParams(use_tc_tiling_on_sc=False),
)
def _sc_gather(idx_hbm, raw_hbm, out_hbm, idx_v, addr_v, buf_v, sem):
    wid = lax.axis_index("s") * _NC + lax.axis_index("c")
    base = wid * _B_PER_W
    pltpu.sync_copy(idx_hbm.at[pl.ds(base, _B_PER_W)], idx_v)

    def addr_body(q, _):
        iv = idx_v[pl.ds(q * _L, _L)]
        s = (iv >> 7) * 1024 + (iv & 127)
        for rr in range(_R * 8):
            r_, rr_ = rr // 8, rr % 8
            addr_v[r_, rr_, pl.ds(q * _L, _L)] = s + (
                (r_ * _TILE_COLS * 8 + rr_) * 128
            )
        return ()

    lax.fori_loop(0, _B_PER_W // _L, addr_body, ())

    copies = []
    for rr in range(_R * 8):
        r_, rr_ = rr // 8, rr % 8
        for k in range(_B_PER_W // 128):
            cp = pltpu.make_async_copy(
                raw_hbm.at[addr_v.at[r_, rr_, pl.ds(k * 128, 128)]],
                buf_v.at[r_, rr_, pl.ds(k * 128, 128)],
                sem,
            )
            cp.start()
            copies.append(cp)
    for cp in copies:
        cp.wait()
    pltpu.sync_copy(buf_v, out_hbm.at[:, :, pl.ds(base, _B_PER_W)])


def kernel(idx, table):
    raw = _sc_relabel(table.T)
    out3 = _sc_gather(idx.astype(jnp.int32), raw.reshape(-1))
    return out3.reshape(EMB_D, B).T


# speedup vs baseline: 23.7266x; 1.0113x over previous
import functools

import jax
import jax.numpy as jnp
from jax import lax
from jax.experimental import pallas as pl
from jax.experimental.pallas import tpu as pltpu
from jax.experimental.pallas import tpu_sc as plsc

EMB_N = 1000000
EMB_D = 32
B = 16384

_NC = 2
_NS = 16
_NW = _NC * _NS
_B_PER_W = B // _NW

_TILE_COLS = (EMB_N + 127) // 128
_R = EMB_D // 8

_W_PER_R = _NW // _R
_COLS_PER_W = -(-_TILE_COLS // _W_PER_R)
_CHUNK = 16


@functools.partial(
    pl.kernel,
    out_type=jax.ShapeDtypeStruct((_R, _TILE_COLS, 8, 128), jnp.float32),
    mesh=plsc.VectorSubcoreMesh(core_axis_name="c", subcore_axis_name="s"),
    scratch_types=[
        pltpu.VMEM((2, _CHUNK, 8, 128), jnp.float32),
        pltpu.SemaphoreType.DMA,
        pltpu.SemaphoreType.DMA,
    ],
    compiler_params=pltpu.CompilerParams(use_tc_tiling_on_sc=True),
)
def _sc_relabel(tableT_hbm, raw_hbm, buf_v, sem_r, sem_w):
    wid = lax.axis_index("s") * _NC + lax.axis_index("c")
    r = wid // _W_PER_R
    c_lo = (wid % _W_PER_R) * _COLS_PER_W
    c_hi = jnp.minimum(c_lo + _COLS_PER_W, _TILE_COLS)
    n_full = (c_hi - c_lo) // _CHUNK

    def tile_read(s, parity, j):
        c0 = c_lo + s * _CHUNK
        return pltpu.make_async_copy(
            tableT_hbm.at[
                pl.ds(r * 8, 8),
                pl.ds(pl.multiple_of((c0 + j) * 128, 128), 128),
            ],
            buf_v.at[parity, j],
            sem_r,
        )

    def chunk_write(s, parity):
        c0 = c_lo + s * _CHUNK
        return pltpu.make_async_copy(
            buf_v.at[parity],
            raw_hbm.at[r, pl.ds(c0, _CHUNK)],
            sem_w,
        )

    @pl.when(n_full > 0)
    def _prime():
        for j in range(_CHUNK):
            tile_read(0, 0, j).start()

    def step(s, _):
        parity = lax.rem(s, 2)

        @pl.when(s >= 2)
        def _drain():
            chunk_write(s - 2, parity).wait()

        for j in range(_CHUNK):
            tile_read(s, parity, j).wait()

        @pl.when(s + 1 < n_full)
        def _next():
            for j in range(_CHUNK):
                tile_read(s + 1, 1 - parity, j).start()

        chunk_write(s, parity).start()
        return ()

    lax.fori_loop(0, n_full, step, ())

    @pl.when(n_full >= 1)
    def _drain_last():
        chunk_write(n_full - 1, lax.rem(n_full - 1, 2)).wait()

    @pl.when(n_full >= 2)
    def _drain_prev():
        chunk_write(n_full - 2, lax.rem(n_full - 2, 2)).wait()

    def col_body(c, _):
        pltpu.sync_copy(
            tableT_hbm.at[
                pl.ds(r * 8, 8),
                pl.ds(pl.multiple_of(c * 128, 128), 128),
            ],
            buf_v.at[0, 0],
        )
        pltpu.sync_copy(
            buf_v.at[0, 0],
            raw_hbm.at[r, c],
        )
        return ()

    lax.fori_loop(c_lo + n_full * _CHUNK, c_hi, col_body, ())


_L = 16


@functools.partial(
    pl.kernel,
    out_type=jax.ShapeDtypeStruct((_R, 8, B), jnp.float32),
    mesh=plsc.VectorSubcoreMesh(core_axis_name="c", subcore_axis_name="s"),
    scratch_types=[
        pltpu.VMEM((_B_PER_W,), jnp.int32),
        pltpu.VMEM((_R, 8, _B_PER_W), jnp.int32),
        pltpu.VMEM((_R, 8, _B_PER_W), jnp.float32),
        pltpu.SemaphoreType.DMA,
    ],
    compiler_params=pltpu.CompilerParams(use_tc_tiling_on_sc=False),
)
def _sc_gather(idx_hbm, raw_hbm, out_hbm, idx_v, addr_v, buf_v, sem):
    wid = lax.axis_index("s") * _NC + lax.axis_index("c")
    base = wid * _B_PER_W
    pltpu.sync_copy(idx_hbm.at[pl.ds(base, _B_PER_W)], idx_v)

    def addr_body(q, _):
        iv = idx_v[pl.ds(q * _L, _L)]
        s = (iv >> 7) * 1024 + (iv & 127)
        for rr in range(_R * 8):
            r_, rr_ = rr // 8, rr % 8
            addr_v[r_, rr_, pl.ds(q * _L, _L)] = s + (
                (r_ * _TILE_COLS * 8 + rr_) * 128
            )
        return ()

    lax.fori_loop(0, _B_PER_W // _L, addr_body, ())

    copies = []
    for rr in range(_R * 8):
        r_, rr_ = rr // 8, rr % 8
        for k in range(_B_PER_W // 128):
            cp = pltpu.make_async_copy(
                raw_hbm.at[addr_v.at[r_, rr_, pl.ds(k * 128, 128)]],
                buf_v.at[r_, rr_, pl.ds(k * 128, 128)],
                sem,
            )
            cp.start()
            copies.append(cp)
    for cp in copies:
        cp.wait()
    pltpu.sync_copy(buf_v, out_hbm.at[:, :, pl.ds(base, _B_PER_W)])


def kernel(idx, table):
    raw = _sc_relabel(table.T)
    out3 = _sc_gather(idx.astype(jnp.int32), raw.reshape(-1))
    return out3.reshape(EMB_D, B).T


# speedup vs baseline: 24.0927x; 1.0154x over previous
import functools

import jax
import jax.numpy as jnp
from jax import lax
from jax.experimental import pallas as pl
from jax.experimental.pallas import tpu as pltpu
from jax.experimental.pallas import tpu_sc as plsc

EMB_N = 1000000
EMB_D = 32
B = 16384

_NC = 2
_NS = 16
_NW = _NC * _NS
_B_PER_W = B // _NW

_TILE_COLS = (EMB_N + 127) // 128
_R = EMB_D // 8

_W_PER_R = _NW // _R
_COLS_PER_W = -(-_TILE_COLS // _W_PER_R)
_CHUNK = 16


@functools.partial(
    pl.kernel,
    out_type=jax.ShapeDtypeStruct((_R, _TILE_COLS, 8, 128), jnp.float32),
    mesh=plsc.VectorSubcoreMesh(core_axis_name="c", subcore_axis_name="s"),
    scratch_types=[
        pltpu.VMEM((2, _CHUNK, 8, 128), jnp.float32),
        pltpu.SemaphoreType.DMA,
        pltpu.SemaphoreType.DMA,
    ],
    compiler_params=pltpu.CompilerParams(use_tc_tiling_on_sc=True),
)
def _sc_relabel(tableT_hbm, raw_hbm, buf_v, sem_r, sem_w):
    wid = lax.axis_index("s") * _NC + lax.axis_index("c")
    r = wid // _W_PER_R
    c_lo = (wid % _W_PER_R) * _COLS_PER_W
    c_hi = jnp.minimum(c_lo + _COLS_PER_W, _TILE_COLS)
    n_full = (c_hi - c_lo) // _CHUNK

    def tile_read(s, parity, j):
        c0 = c_lo + s * _CHUNK
        return pltpu.make_async_copy(
            tableT_hbm.at[
                pl.ds(r * 8, 8),
                pl.ds(pl.multiple_of((c0 + j) * 128, 128), 128),
            ],
            buf_v.at[parity, j],
            sem_r,
        )

    def chunk_write(s, parity):
        c0 = c_lo + s * _CHUNK
        return pltpu.make_async_copy(
            buf_v.at[parity],
            raw_hbm.at[r, pl.ds(c0, _CHUNK)],
            sem_w,
        )

    @pl.when(n_full > 0)
    def _prime():
        for j in range(_CHUNK):
            tile_read(0, 0, j).start()

    def step(s, _):
        parity = lax.rem(s, 2)

        @pl.when(s >= 2)
        def _drain():
            chunk_write(s - 2, parity).wait()

        for j in range(_CHUNK):
            tile_read(s, parity, j).wait()

        @pl.when(s + 1 < n_full)
        def _next():
            for j in range(_CHUNK):
                tile_read(s + 1, 1 - parity, j).start()

        chunk_write(s, parity).start()
        return ()

    lax.fori_loop(0, n_full, step, ())

    @pl.when(n_full >= 1)
    def _drain_last():
        chunk_write(n_full - 1, lax.rem(n_full - 1, 2)).wait()

    @pl.when(n_full >= 2)
    def _drain_prev():
        chunk_write(n_full - 2, lax.rem(n_full - 2, 2)).wait()

    def col_body(c, _):
        pltpu.sync_copy(
            tableT_hbm.at[
                pl.ds(r * 8, 8),
                pl.ds(pl.multiple_of(c * 128, 128), 128),
            ],
            buf_v.at[0, 0],
        )
        pltpu.sync_copy(
            buf_v.at[0, 0],
            raw_hbm.at[r, c],
        )
        return ()

    lax.fori_loop(c_lo + n_full * _CHUNK, c_hi, col_body, ())


_L = 16


@functools.partial(
    pl.kernel,
    out_type=jax.ShapeDtypeStruct((_R, 8, B), jnp.float32),
    mesh=plsc.VectorSubcoreMesh(core_axis_name="c", subcore_axis_name="s"),
    scratch_types=[
        pltpu.VMEM((_B_PER_W,), jnp.int32),
        pltpu.VMEM((_R, 8, _B_PER_W), jnp.int32),
        pltpu.VMEM((_R, 8, _B_PER_W), jnp.float32),
        pltpu.SemaphoreType.DMA,
    ],
    compiler_params=pltpu.CompilerParams(use_tc_tiling_on_sc=False),
)
def _sc_gather(idx_hbm, raw_hbm, out_hbm, idx_v, addr_v, buf_v, sem):
    wid = lax.axis_index("s") * _NC + lax.axis_index("c")
    base = wid * _B_PER_W
    pltpu.sync_copy(idx_hbm.at[pl.ds(base, _B_PER_W)], idx_v)

    def addr_body(q, _):
        iv = idx_v[pl.ds(q * _L, _L)]
        s = (iv >> 7) * 1024 + (iv & 127)
        for rr in range(_R * 8):
            r_, rr_ = rr // 8, rr % 8
            addr_v[r_, rr_, pl.ds(q * _L, _L)] = s + (
                (r_ * _TILE_COLS * 8 + rr_) * 128
            )
        return ()

    lax.fori_loop(0, _B_PER_W // _L, addr_body, ())

    copies = []
    for rr in range(_R * 8):
        r_, rr_ = rr // 8, rr % 8
        cp = pltpu.make_async_copy(
            raw_hbm.at[addr_v.at[r_, rr_]],
            buf_v.at[r_, rr_],
            sem,
        )
        cp.start()
        copies.append(cp)
    for cp in copies:
        cp.wait()
    pltpu.sync_copy(buf_v, out_hbm.at[:, :, pl.ds(base, _B_PER_W)])


def kernel(idx, table):
    raw = _sc_relabel(table.T)
    out3 = _sc_gather(idx.astype(jnp.int32), raw.reshape(-1))
    return out3.reshape(EMB_D, B).T


# speedup vs baseline: 26.6312x; 1.1054x over previous
import functools

import jax
import jax.numpy as jnp
from jax import lax
from jax.experimental import pallas as pl
from jax.experimental.pallas import tpu as pltpu
from jax.experimental.pallas import tpu_sc as plsc

EMB_N = 1000000
EMB_D = 32
B = 16384

_NC = 2
_NS = 16
_NW = _NC * _NS
_B_PER_W = B // _NW

_TILE_COLS = (EMB_N + 127) // 128
_R = EMB_D // 8

_W_PER_R = _NW // _R
_COLS_PER_W = -(-_TILE_COLS // _W_PER_R)
_CHUNK = 32


@functools.partial(
    pl.kernel,
    out_type=jax.ShapeDtypeStruct((_R, _TILE_COLS, 8, 128), jnp.float32),
    mesh=plsc.VectorSubcoreMesh(core_axis_name="c", subcore_axis_name="s"),
    scratch_types=[
        pltpu.VMEM((2, _CHUNK, 8, 128), jnp.float32),
        pltpu.SemaphoreType.DMA,
        pltpu.SemaphoreType.DMA,
    ],
    compiler_params=pltpu.CompilerParams(use_tc_tiling_on_sc=True),
)
def _sc_relabel(tableT_hbm, raw_hbm, buf_v, sem_r, sem_w):
    wid = lax.axis_index("s") * _NC + lax.axis_index("c")
    r = wid // _W_PER_R
    c_lo = (wid % _W_PER_R) * _COLS_PER_W
    c_hi = jnp.minimum(c_lo + _COLS_PER_W, _TILE_COLS)
    n_full = (c_hi - c_lo) // _CHUNK

    def tile_read(s, parity, j):
        c0 = c_lo + s * _CHUNK
        return pltpu.make_async_copy(
            tableT_hbm.at[
                pl.ds(r * 8, 8),
                pl.ds(pl.multiple_of((c0 + j) * 128, 128), 128),
            ],
            buf_v.at[parity, j],
            sem_r,
        )

    def chunk_write(s, parity):
        c0 = c_lo + s * _CHUNK
        return pltpu.make_async_copy(
            buf_v.at[parity],
            raw_hbm.at[r, pl.ds(c0, _CHUNK)],
            sem_w,
        )

    @pl.when(n_full > 0)
    def _prime():
        for j in range(_CHUNK):
            tile_read(0, 0, j).start()

    def step(s, _):
        parity = lax.rem(s, 2)

        @pl.when(s >= 2)
        def _drain():
            chunk_write(s - 2, parity).wait()

        for j in range(_CHUNK):
            tile_read(s, parity, j).wait()

        @pl.when(s + 1 < n_full)
        def _next():
            for j in range(_CHUNK):
                tile_read(s + 1, 1 - parity, j).start()

        chunk_write(s, parity).start()
        return ()

    lax.fori_loop(0, n_full, step, ())

    @pl.when(n_full >= 1)
    def _drain_last():
        chunk_write(n_full - 1, lax.rem(n_full - 1, 2)).wait()

    @pl.when(n_full >= 2)
    def _drain_prev():
        chunk_write(n_full - 2, lax.rem(n_full - 2, 2)).wait()

    def col_body(c, _):
        pltpu.sync_copy(
            tableT_hbm.at[
                pl.ds(r * 8, 8),
                pl.ds(pl.multiple_of(c * 128, 128), 128),
            ],
            buf_v.at[0, 0],
        )
        pltpu.sync_copy(
            buf_v.at[0, 0],
            raw_hbm.at[r, c],
        )
        return ()

    lax.fori_loop(c_lo + n_full * _CHUNK, c_hi, col_body, ())


_L = 16


@functools.partial(
    pl.kernel,
    out_type=jax.ShapeDtypeStruct((_R, 8, B), jnp.float32),
    mesh=plsc.VectorSubcoreMesh(core_axis_name="c", subcore_axis_name="s"),
    scratch_types=[
        pltpu.VMEM((_B_PER_W,), jnp.int32),
        pltpu.VMEM((_R, 8, _B_PER_W), jnp.int32),
        pltpu.VMEM((_R, 8, _B_PER_W), jnp.float32),
        pltpu.SemaphoreType.DMA,
    ],
    compiler_params=pltpu.CompilerParams(use_tc_tiling_on_sc=False),
)
def _sc_gather(idx_hbm, raw_hbm, out_hbm, idx_v, addr_v, buf_v, sem):
    wid = lax.axis_index("s") * _NC + lax.axis_index("c")
    base = wid * _B_PER_W
    pltpu.sync_copy(idx_hbm.at[pl.ds(base, _B_PER_W)], idx_v)

    def addr_body(q, _):
        iv = idx_v[pl.ds(q * _L, _L)]
        s = (iv >> 7) * 1024 + (iv & 127)
        for rr in range(_R * 8):
            r_, rr_ = rr // 8, rr % 8
            addr_v[r_, rr_, pl.ds(q * _L, _L)] = s + (
                (r_ * _TILE_COLS * 8 + rr_) * 128
            )
        return ()

    lax.fori_loop(0, _B_PER_W // _L, addr_body, ())

    copies = []
    for rr in range(_R * 8):
        r_, rr_ = rr // 8, rr % 8
        cp = pltpu.make_async_copy(
            raw_hbm.at[addr_v.at[r_, rr_]],
            buf_v.at[r_, rr_],
            sem,
        )
        cp.start()
        copies.append(cp)
    for cp in copies:
        cp.wait()
    pltpu.sync_copy(buf_v, out_hbm.at[:, :, pl.ds(base, _B_PER_W)])


def kernel(idx, table):
    raw = _sc_relabel(table.T)
    out3 = _sc_gather(idx.astype(jnp.int32), raw.reshape(-1))
    return out3.reshape(EMB_D, B).T


# speedup vs baseline: 27.1385x; 1.0190x over previous
import functools

import jax
import jax.numpy as jnp
from jax import lax
from jax.experimental import pallas as pl
from jax.experimental.pallas import tpu as pltpu
from jax.experimental.pallas import tpu_sc as plsc

EMB_N = 1000000
EMB_D = 32
B = 16384

_NC = 2
_NS = 16
_NW = _NC * _NS
_B_PER_W = B // _NW

_TILE_COLS = (EMB_N + 127) // 128
_R = EMB_D // 8

_W_PER_R = _NW // _R
_COLS_PER_W = -(-_TILE_COLS // _W_PER_R)
_CHUNK = 48


@functools.partial(
    pl.kernel,
    out_type=jax.ShapeDtypeStruct((_R, _TILE_COLS, 8, 128), jnp.float32),
    mesh=plsc.VectorSubcoreMesh(core_axis_name="c", subcore_axis_name="s"),
    scratch_types=[
        pltpu.VMEM((2, _CHUNK, 8, 128), jnp.float32),
        pltpu.SemaphoreType.DMA,
        pltpu.SemaphoreType.DMA,
    ],
    compiler_params=pltpu.CompilerParams(use_tc_tiling_on_sc=True),
)
def _sc_relabel(tableT_hbm, raw_hbm, buf_v, sem_r, sem_w):
    wid = lax.axis_index("s") * _NC + lax.axis_index("c")
    r = wid // _W_PER_R
    c_lo = (wid % _W_PER_R) * _COLS_PER_W
    c_hi = jnp.minimum(c_lo + _COLS_PER_W, _TILE_COLS)
    n_full = (c_hi - c_lo) // _CHUNK

    def tile_read(s, parity, j):
        c0 = c_lo + s * _CHUNK
        return pltpu.make_async_copy(
            tableT_hbm.at[
                pl.ds(r * 8, 8),
                pl.ds(pl.multiple_of((c0 + j) * 128, 128), 128),
            ],
            buf_v.at[parity, j],
            sem_r,
        )

    def chunk_write(s, parity):
        c0 = c_lo + s * _CHUNK
        return pltpu.make_async_copy(
            buf_v.at[parity],
            raw_hbm.at[r, pl.ds(c0, _CHUNK)],
            sem_w,
        )

    @pl.when(n_full > 0)
    def _prime():
        for j in range(_CHUNK):
            tile_read(0, 0, j).start()

    def step(s, _):
        parity = lax.rem(s, 2)

        @pl.when(s >= 2)
        def _drain():
            chunk_write(s - 2, parity).wait()

        for j in range(_CHUNK):
            tile_read(s, parity, j).wait()

        @pl.when(s + 1 < n_full)
        def _next():
            for j in range(_CHUNK):
                tile_read(s + 1, 1 - parity, j).start()

        chunk_write(s, parity).start()
        return ()

    lax.fori_loop(0, n_full, step, ())

    @pl.when(n_full >= 1)
    def _drain_last():
        chunk_write(n_full - 1, lax.rem(n_full - 1, 2)).wait()

    @pl.when(n_full >= 2)
    def _drain_prev():
        chunk_write(n_full - 2, lax.rem(n_full - 2, 2)).wait()

    def col_body(c, _):
        pltpu.sync_copy(
            tableT_hbm.at[
                pl.ds(r * 8, 8),
                pl.ds(pl.multiple_of(c * 128, 128), 128),
            ],
            buf_v.at[0, 0],
        )
        pltpu.sync_copy(
            buf_v.at[0, 0],
            raw_hbm.at[r, c],
        )
        return ()

    lax.fori_loop(c_lo + n_full * _CHUNK, c_hi, col_body, ())


_L = 16


@functools.partial(
    pl.kernel,
    out_type=jax.ShapeDtypeStruct((_R, 8, B), jnp.float32),
    mesh=plsc.VectorSubcoreMesh(core_axis_name="c", subcore_axis_name="s"),
    scratch_types=[
        pltpu.VMEM((_B_PER_W,), jnp.int32),
        pltpu.VMEM((_R, 8, _B_PER_W), jnp.int32),
        pltpu.VMEM((_R, 8, _B_PER_W), jnp.float32),
        pltpu.SemaphoreType.DMA,
    ],
    compiler_params=pltpu.CompilerParams(use_tc_tiling_on_sc=False),
)
def _sc_gather(idx_hbm, raw_hbm, out_hbm, idx_v, addr_v, buf_v, sem):
    wid = lax.axis_index("s") * _NC + lax.axis_index("c")
    base = wid * _B_PER_W
    pltpu.sync_copy(idx_hbm.at[pl.ds(base, _B_PER_W)], idx_v)

    def addr_body(q, _):
        iv = idx_v[pl.ds(q * _L, _L)]
        s = (iv >> 7) * 1024 + (iv & 127)
        for rr in range(_R * 8):
            r_, rr_ = rr // 8, rr % 8
            addr_v[r_, rr_, pl.ds(q * _L, _L)] = s + (
                (r_ * _TILE_COLS * 8 + rr_) * 128
            )
        return ()

    lax.fori_loop(0, _B_PER_W // _L, addr_body, ())

    copies = []
    for rr in range(_R * 8):
        r_, rr_ = rr // 8, rr % 8
        cp = pltpu.make_async_copy(
            raw_hbm.at[addr_v.at[r_, rr_]],
            buf_v.at[r_, rr_],
            sem,
        )
        cp.start()
        copies.append(cp)
    for cp in copies:
        cp.wait()
    pltpu.sync_copy(buf_v, out_hbm.at[:, :, pl.ds(base, _B_PER_W)])


def kernel(idx, table):
    raw = _sc_relabel(table.T)
    out3 = _sc_gather(idx.astype(jnp.int32), raw.reshape(-1))
    return out3.reshape(EMB_D, B).T
